# baseline (speedup 1.0000x reference)
import functools

import jax
import jax.numpy as jnp
from jax import lax
from jax.experimental import pallas as pl
from jax.experimental.pallas import tpu as pltpu
from jax.experimental.pallas import tpu_sc as plsc

E = 64
D = 768
FF = 3072
T = 2048
CHUNK = 512
RCH = 256


def _router_body(x_ref, wg_ref, logits_ref, p_ref, order_ref, starts_ref,
                 ends_ref):
    x = x_ref[...]
    wg = wg_ref[...]
    logits = lax.dot_general(x, wg, (((1,), (1,)), ((), ())),
                             preferred_element_type=jnp.float32)
    logits_ref[...] = logits
    m = jnp.max(logits, axis=1, keepdims=True)
    eids = lax.broadcasted_iota(jnp.int32, (T, E), 1)
    idx = jnp.min(jnp.where(logits == m, eids, E), axis=1, keepdims=True)
    onehot = (idx == eids).astype(jnp.float32)
    counts = jnp.sum(onehot, axis=0, keepdims=True)
    r_i = lax.broadcasted_iota(jnp.int32, (E, E), 0)
    c_i = lax.broadcasted_iota(jnp.int32, (E, E), 1)
    excl = (r_i < c_i).astype(jnp.float32)
    counts_hi = jnp.floor(counts * (1.0 / 256.0))
    counts_lo = counts - 256.0 * counts_hi
    dims_oe = (((1,), (0,)), ((), ()))
    offsets = (256.0 * lax.dot_general(counts_hi, excl, dims_oe,
                                       preferred_element_type=jnp.float32)
               + lax.dot_general(counts_lo, excl, dims_oe,
                                 preferred_element_type=jnp.float32))
    starts_ref[...] = (offsets + 0.5).astype(jnp.int32)
    ends_ref[...] = (offsets + counts + 0.5).astype(jnp.int32)
    rr = lax.broadcasted_iota(jnp.int32, (RCH, RCH), 0)
    cc = lax.broadcasted_iota(jnp.int32, (RCH, RCH), 1)
    tril = (cc < rr).astype(jnp.float32)
    carry = jnp.zeros((1, E), jnp.float32)
    pos_chunks = []
    for i in range(T // RCH):
        oh = onehot[i * RCH:(i + 1) * RCH]
        rank = lax.dot_general(tril, oh, (((1,), (0,)), ((), ())),
                               preferred_element_type=jnp.float32) + carry
        carry = carry + jnp.sum(oh, axis=0, keepdims=True)
        pos = jnp.sum(oh * (rank + offsets), axis=1, keepdims=True)
        p_ref[pl.ds(i * RCH, RCH), :] = (pos + 0.5).astype(jnp.int32)
        pos_chunks.append(pos)
    pos_all = jnp.floor(jnp.concatenate(pos_chunks, axis=0) + 0.5)
    tids_i = lax.broadcasted_iota(jnp.int32, (T, 1), 0)
    tids_hi = (tids_i // 256).astype(jnp.float32)
    tids_lo = (tids_i % 256).astype(jnp.float32)
    dims_t = (((0,), (0,)), ((), ()))
    for j in range(T // RCH):
        jv = jnp.float32(j * RCH) + lax.broadcasted_iota(
            jnp.int32, (T, RCH), 1).astype(jnp.float32)
        match = (pos_all == jv).astype(jnp.float32)
        ob = (256.0 * lax.dot_general(match, tids_hi, dims_t,
                                      preferred_element_type=jnp.float32)
              + lax.dot_general(match, tids_lo, dims_t,
                                preferred_element_type=jnp.float32))
        order_ref[pl.ds(j * RCH, RCH), :] = (ob + 0.5).astype(jnp.int32)


FSPLIT = 2


def _expert_body(starts_ref, ends_ref, sx_ref, w1_ref, b1_ref, w2_ref, b2_ref,
                 h_ref):
    e = pl.program_id(0)
    f = pl.program_id(1)

    @pl.when((e == 0) & (f == 0))
    def _():
        h_ref[...] = jnp.zeros_like(h_ref)

    start = starts_ref[0, e]
    end = ends_ref[0, e]
    w1 = w1_ref[0].astype(jnp.bfloat16)
    w2 = w2_ref[0].astype(jnp.bfloat16)
    b1 = b1_ref[0]
    b2 = b2_ref[0] * (f == 0).astype(jnp.float32)
    c0 = start // CHUNK
    c1 = (end + CHUNK - 1) // CHUNK

    def body(c, carry):
        cb = c * CHUNK
        xs = sx_ref[pl.ds(cb, CHUNK), :].astype(jnp.bfloat16)
        h1 = lax.dot_general(xs, w1, (((1,), (1,)), ((), ())),
                             preferred_element_type=jnp.float32)
        h1 = jnp.maximum(h1 + b1, 0.0).astype(jnp.bfloat16)
        h2 = lax.dot_general(h1, w2, (((1,), (1,)), ((), ())),
                             preferred_element_type=jnp.float32)
        h2 = h2 + b2
        rows = cb + lax.broadcasted_iota(jnp.int32, (CHUNK, 1), 0)
        msk = (rows >= start) & (rows < end)
        h_ref[pl.ds(cb, CHUNK), :] = (
            h_ref[pl.ds(cb, CHUNK), :] + jnp.where(msk, h2, 0.0))
        return carry

    @pl.when(c1 > c0)
    def _():
        body(c0, 0)

    @pl.when(c1 > c0 + 1)
    def _():
        body(c0 + 1, 0)

    lax.fori_loop(c0 + 2, c1, body, 0)


def _make_router():
    return pl.pallas_call(
        _router_body,
        out_shape=(
            jax.ShapeDtypeStruct((T, E), jnp.float32),
            jax.ShapeDtypeStruct((T, 1), jnp.int32),
            jax.ShapeDtypeStruct((T, 1), jnp.int32),
            jax.ShapeDtypeStruct((1, E), jnp.int32),
            jax.ShapeDtypeStruct((1, E), jnp.int32),
        ),
    )


def _make_experts():
    grid = (E, FSPLIT)
    ffb = FF // FSPLIT
    return pl.pallas_call(
        _expert_body,
        grid=grid,
        in_specs=[
            pl.BlockSpec(memory_space=pltpu.SMEM),
            pl.BlockSpec(memory_space=pltpu.SMEM),
            pl.BlockSpec((T, D), lambda e, f: (0, 0)),
            pl.BlockSpec((1, ffb, D), lambda e, f: (e, f, 0)),
            pl.BlockSpec((1, 1, ffb), lambda e, f: (e, 0, f)),
            pl.BlockSpec((1, D, ffb), lambda e, f: (e, 0, f)),
            pl.BlockSpec((1, 1, D), lambda e, f: (e, 0, 0)),
        ],
        out_specs=pl.BlockSpec((T, D), lambda e, f: (0, 0)),
        out_shape=jax.ShapeDtypeStruct((T, D), jnp.float32),
    )


def _make_sc_gather():
    info = plsc.get_sparse_core_info()
    nw = info.num_cores * info.num_subcores
    rows_per_w = T // nw
    mesh = plsc.VectorSubcoreMesh(core_axis_name="c", subcore_axis_name="s")

    @functools.partial(
        pl.kernel,
        mesh=mesh,
        out_type=jax.ShapeDtypeStruct((T, D), jnp.float32),
        scratch_types=[
            pltpu.VMEM((rows_per_w,), jnp.int32),
            pltpu.VMEM((rows_per_w, D), jnp.float32),
            pltpu.SemaphoreType.DMA,
        ],
    )
    def gather(table_hbm, idx_hbm, out_hbm, idx_v, rows_v, sem):
        wid = lax.axis_index("s") * info.num_cores + lax.axis_index("c")
        base = wid * rows_per_w
        pltpu.sync_copy(idx_hbm.at[pl.ds(base, rows_per_w)], idx_v)
        pltpu.async_copy(table_hbm.at[idx_v], rows_v, sem).wait()
        pltpu.sync_copy(rows_v, out_hbm.at[pl.ds(base, rows_per_w)])

    return gather


def kernel(hidden_states, Wg, W1, b1, W2, b2):
    orig_shape = hidden_states.shape
    x = hidden_states.reshape(T, D)
    logits, p, order, starts, ends = _make_router()(x, Wg)
    gather = _make_sc_gather()
    sorted_x = gather(x, order.reshape(T))
    h = _make_experts()(starts, ends, sorted_x, W1, b1.reshape(E, 1, FF), W2,
                        b2.reshape(E, 1, D))
    out = gather(h, p.reshape(T))
    return out.reshape(orig_shape), logits


# speedup vs baseline: 1.1548x; 1.1548x over previous
import functools

import jax
import jax.numpy as jnp
from jax import lax
from jax.experimental import pallas as pl
from jax.experimental.pallas import tpu as pltpu
from jax.experimental.pallas import tpu_sc as plsc

E = 64
D = 768
FF = 3072
T = 2048
CHUNK = 256
RCH = 256


def _router_body(x_ref, wg_ref, logits_ref, p_ref, order_ref, starts_ref,
                 ends_ref):
    x = x_ref[...]
    wg = wg_ref[...]
    logits = lax.dot_general(x, wg, (((1,), (1,)), ((), ())),
                             preferred_element_type=jnp.float32)
    logits_ref[...] = logits
    m = jnp.max(logits, axis=1, keepdims=True)
    eids = lax.broadcasted_iota(jnp.int32, (T, E), 1)
    idx = jnp.min(jnp.where(logits == m, eids, E), axis=1, keepdims=True)
    onehot = (idx == eids).astype(jnp.float32)
    counts = jnp.sum(onehot, axis=0, keepdims=True)
    r_i = lax.broadcasted_iota(jnp.int32, (E, E), 0)
    c_i = lax.broadcasted_iota(jnp.int32, (E, E), 1)
    excl = (r_i < c_i).astype(jnp.float32)
    counts_hi = jnp.floor(counts * (1.0 / 256.0))
    counts_lo = counts - 256.0 * counts_hi
    dims_oe = (((1,), (0,)), ((), ()))
    offsets = (256.0 * lax.dot_general(counts_hi, excl, dims_oe,
                                       preferred_element_type=jnp.float32)
               + lax.dot_general(counts_lo, excl, dims_oe,
                                 preferred_element_type=jnp.float32))
    starts_ref[...] = (offsets + 0.5).astype(jnp.int32)
    ends_ref[...] = (offsets + counts + 0.5).astype(jnp.int32)
    rr = lax.broadcasted_iota(jnp.int32, (RCH, RCH), 0)
    cc = lax.broadcasted_iota(jnp.int32, (RCH, RCH), 1)
    tril = (cc < rr).astype(jnp.float32)
    carry = jnp.zeros((1, E), jnp.float32)
    pos_chunks = []
    for i in range(T // RCH):
        oh = onehot[i * RCH:(i + 1) * RCH]
        rank = lax.dot_general(tril, oh, (((1,), (0,)), ((), ())),
                               preferred_element_type=jnp.float32) + carry
        carry = carry + jnp.sum(oh, axis=0, keepdims=True)
        pos = jnp.sum(oh * (rank + offsets), axis=1, keepdims=True)
        p_ref[pl.ds(i * RCH, RCH), :] = (pos + 0.5).astype(jnp.int32)
        pos_chunks.append(pos)
    pos_all = jnp.floor(jnp.concatenate(pos_chunks, axis=0) + 0.5)
    tids_i = lax.broadcasted_iota(jnp.int32, (T, 1), 0)
    tids_hi = (tids_i // 256).astype(jnp.float32)
    tids_lo = (tids_i % 256).astype(jnp.float32)
    dims_t = (((0,), (0,)), ((), ()))
    for j in range(T // RCH):
        jv = jnp.float32(j * RCH) + lax.broadcasted_iota(
            jnp.int32, (T, RCH), 1).astype(jnp.float32)
        match = (pos_all == jv).astype(jnp.float32)
        ob = (256.0 * lax.dot_general(match, tids_hi, dims_t,
                                      preferred_element_type=jnp.float32)
              + lax.dot_general(match, tids_lo, dims_t,
                                preferred_element_type=jnp.float32))
        order_ref[pl.ds(j * RCH, RCH), :] = (ob + 0.5).astype(jnp.int32)


FSPLIT = 2


def _expert_body(starts_ref, ends_ref, sx_ref, w1_ref, b1_ref, w2_ref, b2_ref,
                 h_ref):
    e = pl.program_id(0)
    f = pl.program_id(1)

    @pl.when((e == 0) & (f == 0))
    def _():
        h_ref[...] = jnp.zeros_like(h_ref)

    start = starts_ref[0, e]
    end = ends_ref[0, e]
    w1 = w1_ref[0].astype(jnp.bfloat16)
    w2 = w2_ref[0].astype(jnp.bfloat16)
    b1 = b1_ref[0]
    b2 = b2_ref[0] * (f == 0).astype(jnp.float32)
    c0 = start // CHUNK
    c1 = (end + CHUNK - 1) // CHUNK

    def body(c, carry):
        cb = c * CHUNK
        xs = sx_ref[pl.ds(cb, CHUNK), :].astype(jnp.bfloat16)
        h1 = lax.dot_general(xs, w1, (((1,), (1,)), ((), ())),
                             preferred_element_type=jnp.float32)
        h1 = jnp.maximum(h1 + b1, 0.0).astype(jnp.bfloat16)
        h2 = lax.dot_general(h1, w2, (((1,), (1,)), ((), ())),
                             preferred_element_type=jnp.float32)
        h2 = h2 + b2
        rows = cb + lax.broadcasted_iota(jnp.int32, (CHUNK, 1), 0)
        msk = (rows >= start) & (rows < end)
        h_ref[pl.ds(cb, CHUNK), :] = (
            h_ref[pl.ds(cb, CHUNK), :] + jnp.where(msk, h2, 0.0))
        return carry

    @pl.when(c1 > c0)
    def _():
        body(c0, 0)

    @pl.when(c1 > c0 + 1)
    def _():
        body(c0 + 1, 0)

    lax.fori_loop(c0 + 2, c1, body, 0)


def _make_router():
    return pl.pallas_call(
        _router_body,
        out_shape=(
            jax.ShapeDtypeStruct((T, E), jnp.float32),
            jax.ShapeDtypeStruct((T, 1), jnp.int32),
            jax.ShapeDtypeStruct((T, 1), jnp.int32),
            jax.ShapeDtypeStruct((1, E), jnp.int32),
            jax.ShapeDtypeStruct((1, E), jnp.int32),
        ),
    )


def _make_experts():
    grid = (E, FSPLIT)
    ffb = FF // FSPLIT
    return pl.pallas_call(
        _expert_body,
        grid=grid,
        in_specs=[
            pl.BlockSpec(memory_space=pltpu.SMEM),
            pl.BlockSpec(memory_space=pltpu.SMEM),
            pl.BlockSpec((T, D), lambda e, f: (0, 0)),
            pl.BlockSpec((1, ffb, D), lambda e, f: (e, f, 0)),
            pl.BlockSpec((1, 1, ffb), lambda e, f: (e, 0, f)),
            pl.BlockSpec((1, D, ffb), lambda e, f: (e, 0, f)),
            pl.BlockSpec((1, 1, D), lambda e, f: (e, 0, 0)),
        ],
        out_specs=pl.BlockSpec((T, D), lambda e, f: (0, 0)),
        out_shape=jax.ShapeDtypeStruct((T, D), jnp.float32),
    )


def _make_sc_gather():
    info = plsc.get_sparse_core_info()
    nw = info.num_cores * info.num_subcores
    rows_per_w = T // nw
    mesh = plsc.VectorSubcoreMesh(core_axis_name="c", subcore_axis_name="s")

    @functools.partial(
        pl.kernel,
        mesh=mesh,
        out_type=jax.ShapeDtypeStruct((T, D), jnp.float32),
        scratch_types=[
            pltpu.VMEM((rows_per_w,), jnp.int32),
            pltpu.VMEM((rows_per_w, D), jnp.float32),
            pltpu.SemaphoreType.DMA,
        ],
    )
    def gather(table_hbm, idx_hbm, out_hbm, idx_v, rows_v, sem):
        wid = lax.axis_index("s") * info.num_cores + lax.axis_index("c")
        base = wid * rows_per_w
        pltpu.sync_copy(idx_hbm.at[pl.ds(base, rows_per_w)], idx_v)
        pltpu.async_copy(table_hbm.at[idx_v], rows_v, sem).wait()
        pltpu.sync_copy(rows_v, out_hbm.at[pl.ds(base, rows_per_w)])

    return gather


def kernel(hidden_states, Wg, W1, b1, W2, b2):
    orig_shape = hidden_states.shape
    x = hidden_states.reshape(T, D)
    logits, p, order, starts, ends = _make_router()(x, Wg)
    gather = _make_sc_gather()
    sorted_x = gather(x, order.reshape(T))
    h = _make_experts()(starts, ends, sorted_x, W1, b1.reshape(E, 1, FF), W2,
                        b2.reshape(E, 1, D))
    out = gather(h, p.reshape(T))
    return out.reshape(orig_shape), logits
